# Initial kernel scaffold; baseline (speedup 1.0000x reference)
import functools

import jax
import jax.numpy as jnp
from jax import lax
from jax.experimental import pallas as pl
from jax.experimental.pallas import tpu as pltpu
from jax.experimental.pallas import tpu_sc as plsc

_B, _S = 16384, 50
_N = _B * _S
_D = 64
_NC, _NS = 2, 16
_NW = _NC * _NS
_BPW = _N // _NW
_CH = 128
_NCH = _BPW // _CH


def _sc_gather(idx, table):
    mesh = plsc.VectorSubcoreMesh(core_axis_name="c", subcore_axis_name="s")

    @functools.partial(
        pl.kernel,
        mesh=mesh,
        out_type=jax.ShapeDtypeStruct((_N, _D), jnp.float32),
        scratch_types=[
            pltpu.VMEM((_NCH, _CH), jnp.int32),
            pltpu.VMEM((2, _CH, _D), jnp.float32),
            pltpu.SemaphoreType.DMA,
            pltpu.SemaphoreType.DMA,
        ],
    )
    def k(idx_hbm, table_hbm, out_hbm, idx_v, rows_v, sem0, sem1):
        wid = lax.axis_index("s") * _NC + lax.axis_index("c")
        base = wid * _BPW
        pltpu.sync_copy(idx_hbm.at[wid], idx_v)

        pltpu.async_copy(table_hbm.at[idx_v.at[0]], rows_v.at[0], sem0)

        def body(j, _):
            nxt = lax.rem(j + 1, 2)

            @pl.when(j + 1 < _NCH)
            def _():
                def fire(b, sem):
                    pltpu.async_copy(
                        table_hbm.at[idx_v.at[j + 1]], rows_v.at[b], sem
                    )
                lax.cond(nxt == 0,
                         lambda: fire(0, sem0),
                         lambda: fire(1, sem1))

            def drain(b, sem):
                pltpu.make_async_copy(
                    table_hbm.at[idx_v.at[j]], rows_v.at[b], sem
                ).wait()
                pltpu.sync_copy(
                    rows_v.at[b], out_hbm.at[pl.ds(base + j * _CH, _CH)]
                )
            lax.cond(lax.rem(j, 2) == 0,
                     lambda: drain(0, sem0),
                     lambda: drain(1, sem1))
            return 0

        lax.fori_loop(0, _NCH, body, 0)

    return k(idx, table)


def kernel(x, table):
    idx = x.reshape(_NW, _NCH, _CH).astype(jnp.int32)
    out = _sc_gather(idx, table)
    return out.reshape(_B, _S, _D)


# speedup vs baseline: 1.8392x; 1.8392x over previous
import functools

import jax
import jax.numpy as jnp
from jax import lax
from jax.experimental import pallas as pl
from jax.experimental.pallas import tpu as pltpu
from jax.experimental.pallas import tpu_sc as plsc

_B, _S = 16384, 50
_N = _B * _S
_D = 64
_NC, _NS = 2, 16
_NW = _NC * _NS
_BPW = _N // _NW
_CH = 128
_NCH = _BPW // _CH


def _sc_gather(idx, table):
    mesh = plsc.VectorSubcoreMesh(core_axis_name="c", subcore_axis_name="s")

    @functools.partial(
        pl.kernel,
        mesh=mesh,
        out_type=jax.ShapeDtypeStruct((_N, _D), jnp.float32),
        compiler_params=pltpu.CompilerParams(use_tc_tiling_on_sc=False),
        scratch_types=[
            pltpu.VMEM((_NCH, _CH), jnp.int32),
            pltpu.VMEM((2, _CH, _D), jnp.float32),
            pltpu.SemaphoreType.DMA,
            pltpu.SemaphoreType.DMA,
        ],
    )
    def k(idx_hbm, table_hbm, out_hbm, idx_v, rows_v, sem0, sem1):
        wid = lax.axis_index("s") * _NC + lax.axis_index("c")
        base = wid * _BPW
        pltpu.sync_copy(idx_hbm.at[wid], idx_v)

        pltpu.async_copy(table_hbm.at[idx_v.at[0]], rows_v.at[0], sem0)

        def body(j, _):
            nxt = lax.rem(j + 1, 2)

            @pl.when(j + 1 < _NCH)
            def _():
                def fire(b, sem):
                    pltpu.async_copy(
                        table_hbm.at[idx_v.at[j + 1]], rows_v.at[b], sem
                    )
                lax.cond(nxt == 0,
                         lambda: fire(0, sem0),
                         lambda: fire(1, sem1))

            def drain(b, sem):
                pltpu.make_async_copy(
                    table_hbm.at[idx_v.at[j]], rows_v.at[b], sem
                ).wait()
                pltpu.sync_copy(
                    rows_v.at[b], out_hbm.at[pl.ds(base + j * _CH, _CH)]
                )
            lax.cond(lax.rem(j, 2) == 0,
                     lambda: drain(0, sem0),
                     lambda: drain(1, sem1))
            return 0

        lax.fori_loop(0, _NCH, body, 0)

    return k(idx, table)


def kernel(x, table):
    idx = x.reshape(_NW, _NCH, _CH).astype(jnp.int32)
    out = _sc_gather(idx, table)
    return out.reshape(_B, _S, _D)


# speedup vs baseline: 1.8747x; 1.0193x over previous
import functools

import jax
import jax.numpy as jnp
from jax import lax
from jax.experimental import pallas as pl
from jax.experimental.pallas import tpu as pltpu
from jax.experimental.pallas import tpu_sc as plsc

_B, _S = 16384, 50
_N = _B * _S
_D = 64
_NC, _NS = 2, 16
_NW = _NC * _NS
_BPW = _N // _NW
_CH = 128
_NCH = _BPW // _CH


_NBUF = 8


def _sc_gather(idx, table):
    mesh = plsc.VectorSubcoreMesh(core_axis_name="c", subcore_axis_name="s")

    @functools.partial(
        pl.kernel,
        mesh=mesh,
        out_type=jax.ShapeDtypeStruct((_N, _D), jnp.float32),
        compiler_params=pltpu.CompilerParams(use_tc_tiling_on_sc=False),
        scratch_types=[
            pltpu.VMEM((_NCH, _CH), jnp.int32),
            pltpu.VMEM((_NBUF, _CH, _D), jnp.float32),
            [pltpu.SemaphoreType.DMA] * _NBUF,
        ],
    )
    def k(idx_hbm, table_hbm, out_hbm, idx_v, rows_v, sems):
        wid = lax.axis_index("s") * _NC + lax.axis_index("c")
        base = wid * _BPW
        pltpu.sync_copy(idx_hbm.at[wid], idx_v)

        def fire(j, b):
            pltpu.async_copy(table_hbm.at[idx_v.at[j]], rows_v.at[b], sems[b])

        for b in range(_NBUF):
            fire(b, b)

        def body(g, _):
            j0 = g * _NBUF
            for b in range(_NBUF):
                j = j0 + b
                pltpu.make_async_copy(
                    table_hbm.at[idx_v.at[0]], rows_v.at[b], sems[b]
                ).wait()
                pltpu.sync_copy(
                    rows_v.at[b], out_hbm.at[pl.ds(base + j * _CH, _CH)]
                )

                @pl.when(j + _NBUF < _NCH)
                def _():
                    fire(j + _NBUF, b)
            return 0

        lax.fori_loop(0, _NCH // _NBUF, body, 0)

    return k(idx, table)


def kernel(x, table):
    idx = x.reshape(_NW, _NCH, _CH).astype(jnp.int32)
    out = _sc_gather(idx, table)
    return out.reshape(_B, _S, _D)


# speedup vs baseline: 1.8758x; 1.0006x over previous
import functools

import jax
import jax.numpy as jnp
from jax import lax
from jax.experimental import pallas as pl
from jax.experimental.pallas import tpu as pltpu
from jax.experimental.pallas import tpu_sc as plsc

_B, _S = 16384, 50
_N = _B * _S
_D = 64
_NC, _NS = 2, 16
_NW = _NC * _NS
_BPW = _N // _NW
_CH = 128
_NCH = _BPW // _CH


_NBUF = 8
_H = 4


def _sc_gather(idx, table):
    mesh = plsc.VectorSubcoreMesh(core_axis_name="c", subcore_axis_name="s")

    @functools.partial(
        pl.kernel,
        mesh=mesh,
        out_type=jax.ShapeDtypeStruct((_N, _D), jnp.float32),
        compiler_params=pltpu.CompilerParams(use_tc_tiling_on_sc=False),
        scratch_types=[
            pltpu.VMEM((_NCH, _CH), jnp.int32),
            pltpu.VMEM((_NBUF, _CH, _D), jnp.float32),
            [pltpu.SemaphoreType.DMA] * _NBUF,
            [pltpu.SemaphoreType.DMA] * _NBUF,
        ],
    )
    def k(idx_hbm, table_hbm, out_hbm, idx_v, rows_v, sem_g, sem_s):
        wid = lax.axis_index("s") * _NC + lax.axis_index("c")
        base = wid * _BPW
        pltpu.sync_copy(idx_hbm.at[wid], idx_v)

        def fire_gather(j, b):
            pltpu.async_copy(table_hbm.at[idx_v.at[j]], rows_v.at[b], sem_g[b])

        def wait_gather(b):
            pltpu.make_async_copy(
                table_hbm.at[idx_v.at[0]], rows_v.at[b], sem_g[b]
            ).wait()

        def fire_store(j, b):
            pltpu.async_copy(
                rows_v.at[b], out_hbm.at[pl.ds(base + j * _CH, _CH)], sem_s[b]
            )

        def wait_store(b):
            pltpu.make_async_copy(
                rows_v.at[b], out_hbm.at[pl.ds(base, _CH)], sem_s[b]
            ).wait()

        for b in range(_H):
            fire_gather(b, b)

        def body(g, _):
            j0 = g * _NBUF
            for b in range(_NBUF):
                j = j0 + b
                f = j + _H
                bf = (b + _H) % _NBUF

                @pl.when(f < _NCH)
                def _():
                    @pl.when(f >= _NBUF)
                    def _():
                        wait_store(bf)
                    fire_gather(f, bf)

                wait_gather(b)
                fire_store(j, b)
            return 0

        lax.fori_loop(0, _NCH // _NBUF, body, 0)

        for b in range(_NBUF - _H, _NBUF):
            wait_store(b)

    return k(idx, table)


def kernel(x, table):
    idx = x.reshape(_NW, _NCH, _CH).astype(jnp.int32)
    out = _sc_gather(idx, table)
    return out.reshape(_B, _S, _D)
